# baseline (speedup 1.0000x reference)
import functools

import jax
import jax.numpy as jnp
from jax import lax
from jax.experimental import pallas as pl
from jax.experimental.pallas import tpu as pltpu
from jax.experimental.pallas import tpu_sc as plsc

_NW = 32
_NC = 2
_GB = 4
_SPLITS = ((0, 128), (128, 72))


def _gather_kernel(BATCH, T, D, rows_per_w, n_groups):
    mesh = plsc.VectorSubcoreMesh(core_axis_name="c", subcore_axis_name="s")

    @functools.partial(
        pl.kernel,
        mesh=mesh,
        compiler_params=pltpu.CompilerParams(use_tc_tiling_on_sc=False),
        out_type=jax.ShapeDtypeStruct((BATCH, T, D), jnp.float32),
        scratch_types=[
            pltpu.VMEM((2, _GB, T), jnp.int32),
            pltpu.VMEM((2, _GB, T, D), jnp.float32),
            pltpu.SemaphoreType.DMA,
            pltpu.SemaphoreType.DMA,
            pltpu.SemaphoreType.DMA,
            pltpu.SemaphoreType.DMA,
        ],
    )
    def k(idx_hbm, table_hbm, out_hbm, idx_v, rows_v, sg0, sg1, sw0, sw1):
        wid = lax.axis_index("s") * _NC + lax.axis_index("c")
        w_b0 = wid * rows_per_w
        sem_g = (sg0, sg1)
        sem_w = (sw0, sw1)

        def run_group(g, b, other):
            @pl.when(g >= 2)
            def _():
                pltpu.make_async_copy(
                    rows_v.at[b],
                    out_hbm.at[pl.ds(w_b0 + (g - 2) * _GB, _GB)],
                    sem_w[b],
                ).wait()

            descs = [
                pltpu.async_copy(
                    table_hbm.at[idx_v.at[b, r, pl.ds(off, sz)]],
                    rows_v.at[b, r, pl.ds(off, sz)],
                    sem_g[b],
                )
                for r in range(_GB)
                for off, sz in _SPLITS
            ]

            @pl.when(g + 1 < n_groups)
            def _():
                pltpu.sync_copy(
                    idx_hbm.at[pl.ds(w_b0 + (g + 1) * _GB, _GB)],
                    idx_v.at[other],
                )

            for d in descs:
                d.wait()
            pltpu.async_copy(
                rows_v.at[b],
                out_hbm.at[pl.ds(w_b0 + g * _GB, _GB)],
                sem_w[b],
            )

        pltpu.sync_copy(idx_hbm.at[pl.ds(w_b0, _GB)], idx_v.at[0])

        @pl.loop(0, n_groups, step=2)
        def _(gbase):
            run_group(gbase, 0, 1)
            run_group(gbase + 1, 1, 0)

        for g, b in ((n_groups - 2, 0), (n_groups - 1, 1)):
            pltpu.make_async_copy(
                rows_v.at[b],
                out_hbm.at[pl.ds(w_b0 + g * _GB, _GB)],
                sem_w[b],
            ).wait()

    return k


def kernel(x, table):
    BATCH, T = x.shape
    D = table.shape[1]
    rows_per_w = BATCH // _NW
    n_groups = rows_per_w // _GB
    return _gather_kernel(BATCH, T, D, rows_per_w, n_groups)(
        x.astype(jnp.int32), table
    )


# speedup vs baseline: 1.3256x; 1.3256x over previous
import functools

import jax
import jax.numpy as jnp
from jax import lax
from jax.experimental import pallas as pl
from jax.experimental.pallas import tpu as pltpu
from jax.experimental.pallas import tpu_sc as plsc

_NW = 32
_NC = 2
_GB = 2
_IB = 8
_SPLITS = ((0, 128), (128, 72))
_DP = 128


def _gather_kernel(BATCH, T, rows_per_w, n_groups):
    mesh = plsc.VectorSubcoreMesh(core_axis_name="c", subcore_axis_name="s")

    @functools.partial(
        pl.kernel,
        mesh=mesh,
        compiler_params=pltpu.CompilerParams(use_tc_tiling_on_sc=False),
        out_type=jax.ShapeDtypeStruct((BATCH, T, _DP), jnp.float32),
        scratch_types=[
            pltpu.VMEM((2, _IB, T), jnp.int32),
            pltpu.VMEM((2, _GB, T, 64), jnp.float32),
            pltpu.SemaphoreType.DMA,
            pltpu.SemaphoreType.DMA,
            pltpu.SemaphoreType.DMA,
            pltpu.SemaphoreType.DMA,
        ],
    )
    def k(idx_hbm, table_hbm, out_hbm, idx_v, rows_v, sg0, sg1, sw0, sw1):
        wid = lax.axis_index("s") * _NC + lax.axis_index("c")
        w_b0 = wid * rows_per_w
        sem_g = (sg0, sg1)
        sem_w = (sw0, sw1)
        gp_per_blk = _IB // _GB

        def run_group(g, k_in_blk, pb):
            b = k_in_blk % 2

            @pl.when(g >= 2)
            def _():
                pltpu.make_async_copy(
                    rows_v.at[b],
                    out_hbm.at[pl.ds(w_b0 + (g - 2) * _GB, _GB), :, pl.ds(0, 64)],
                    sem_w[b],
                ).wait()

            descs = [
                pltpu.async_copy(
                    table_hbm.at[idx_v.at[pb, k_in_blk * _GB + r, pl.ds(off, sz)]],
                    rows_v.at[b, r, pl.ds(off, sz)],
                    sem_g[b],
                )
                for r in range(_GB)
                for off, sz in _SPLITS
            ]

            @pl.when((k_in_blk == gp_per_blk - 1) & (g + 1 < n_groups))
            def _():
                pltpu.sync_copy(
                    idx_hbm.at[pl.ds(w_b0 + (g + 1) * _GB, _IB)],
                    idx_v.at[1 - pb],
                )

            for d in descs:
                d.wait()
            pltpu.async_copy(
                rows_v.at[b],
                out_hbm.at[pl.ds(w_b0 + g * _GB, _GB), :, pl.ds(0, 64)],
                sem_w[b],
            )

        pltpu.sync_copy(idx_hbm.at[pl.ds(w_b0, _IB)], idx_v.at[0])

        @pl.loop(0, n_groups, step=gp_per_blk)
        def _(gbase):
            pb = (gbase // gp_per_blk) % 2
            for kk in range(gp_per_blk):
                run_group(gbase + kk, kk, pb)

        for g, b in ((n_groups - 2, 0), (n_groups - 1, 1)):
            pltpu.make_async_copy(
                rows_v.at[b],
                out_hbm.at[pl.ds(w_b0 + g * _GB, _GB), :, pl.ds(0, 64)],
                sem_w[b],
            ).wait()

    return k


def kernel(x, table):
    BATCH, T = x.shape
    rows_per_w = BATCH // _NW
    n_groups = rows_per_w // _GB
    out = _gather_kernel(BATCH, T, rows_per_w, n_groups)(
        x.astype(jnp.int32), table
    )
    return out[:, :, :64]
